# baseline (speedup 1.0000x reference)
import functools

import jax
import jax.numpy as jnp
from jax import lax
from jax.experimental import pallas as pl
from jax.experimental.pallas import tpu as pltpu
from jax.experimental.pallas import tpu_sc as plsc

NC = 2
NS = 16
NW = NC * NS
L = 16

N_NODES = 100_000
ACC = 100_352
CC = 3200
U = 8
ZU = 32


def _sc_partials(edge_index, yh):
    (E,) = yh.shape
    nchunks = E // CC
    assert nchunks * CC == E
    npairs = (nchunks + NW - 1) // NW
    npairs = (npairs + 1) // 2

    mesh = plsc.VectorSubcoreMesh(core_axis_name="c", subcore_axis_name="s")

    @functools.partial(
        pl.kernel,
        out_type=jax.ShapeDtypeStruct((NW, ACC), jnp.float32),
        mesh=mesh,
        scratch_types=[
            pltpu.VMEM((ACC,), jnp.float32),
            pltpu.VMEM((2, CC), jnp.int32),
            pltpu.VMEM((CC,), jnp.float32),
            pltpu.VMEM((2, CC), jnp.int32),
            pltpu.VMEM((CC,), jnp.float32),
            pltpu.SemaphoreType.DMA,
            pltpu.SemaphoreType.DMA,
        ],
        compiler_params=pltpu.CompilerParams(needs_layout_passes=False),
    )
    def k(ei_hbm, y_hbm, out_hbm, acc, eb0, yb0, eb1, yb1, sem0, sem1):
        wid = lax.axis_index("s") * NC + lax.axis_index("c")
        n_c = (nchunks - wid + NW - 1) // NW
        sems = (sem0, sem1)
        bufs = ((eb0, yb0), (eb1, yb1))

        zeros = jnp.zeros((L,), jnp.float32)

        @plsc.parallel_loop(0, ACC // L, unroll=ZU)
        def _(i):
            acc[pl.ds(i * L, L)] = zeros

        def copies(j, b):
            c = wid + j * NW
            coff = pl.multiple_of(c * CC, 128)
            ebb, ybb = bufs[b]
            return (
                pltpu.make_async_copy(
                    ei_hbm.at[:, pl.ds(coff, CC)], ebb, sems[b]),
                pltpu.make_async_copy(
                    y_hbm.at[pl.ds(coff, CC)], ybb, sems[b]),
            )

        def issue(j, b):
            for cp in copies(j, b):
                cp.start()

        def wait(b):
            for cp in copies(0, b):
                cp.wait()

        row0 = jnp.zeros((L,), jnp.int32)
        row1 = jnp.ones((L,), jnp.int32)
        lanes = lax.iota(jnp.int32, L)

        def compute(b):
            ebb, ybb = bufs[b]

            @plsc.parallel_loop(0, CC // L, unroll=U)
            def _(i):
                col = lanes + i * L
                s = plsc.load_gather(ebb, [row0, col])
                d = plsc.load_gather(ebb, [row1, col])
                v = ybb[pl.ds(i * L, L)]
                plsc.addupdate_scatter(acc, [s], v)
                plsc.addupdate_scatter(acc, [d], -v)

        issue(0, 0)

        def pair_body(p, _):
            j0 = p * 2
            j1 = j0 + 1

            @pl.when(j1 < n_c)
            def _():
                issue(j1, 1)

            @pl.when(j0 < n_c)
            def _():
                wait(0)
                compute(0)

            @pl.when(j0 + 2 < n_c)
            def _():
                issue(j0 + 2, 0)

            @pl.when(j1 < n_c)
            def _():
                wait(1)
                compute(1)

            return 0

        lax.fori_loop(0, npairs, pair_body, 0)

        pltpu.sync_copy(acc, out_hbm.at[wid])

    return k(edge_index, yh)


def _tc_loss(partials):
    grid = 14
    blk = ACC // grid

    def body(p_ref, o_ref):
        @pl.when(pl.program_id(0) == 0)
        def _():
            o_ref[0, 0] = 0.0

        d = jnp.sum(p_ref[...], axis=0)
        o_ref[0, 0] += jnp.sum(jnp.abs(d))

    out = pl.pallas_call(
        body,
        grid=(grid,),
        in_specs=[pl.BlockSpec((NW, blk), lambda i: (0, i))],
        out_specs=pl.BlockSpec(memory_space=pltpu.SMEM),
        out_shape=jax.ShapeDtypeStruct((1, 1), jnp.float32),
    )(partials)
    return out[0, 0]


def kernel(edge_index, y_hat):
    yh = jnp.squeeze(y_hat, -1).astype(jnp.float32)
    partials = _sc_partials(edge_index, yh)
    total = _tc_loss(partials)
    return total / yh.shape[0]


# speedup vs baseline: 1.0519x; 1.0519x over previous
import functools

import jax
import jax.numpy as jnp
from jax import lax
from jax.experimental import pallas as pl
from jax.experimental.pallas import tpu as pltpu
from jax.experimental.pallas import tpu_sc as plsc

NC = 2
NS = 16
NW = NC * NS
L = 16

N_NODES = 100_000
ACC = 100_352
CC = 3200
U = 8
ZU = 32


def _sc_partials(edge_index, yh):
    (E,) = yh.shape
    nchunks = E // CC
    assert nchunks * CC == E
    npairs = (nchunks + NW - 1) // NW
    npairs = (npairs + 1) // 2

    mesh = plsc.VectorSubcoreMesh(core_axis_name="c", subcore_axis_name="s")

    @functools.partial(
        pl.kernel,
        out_type=jax.ShapeDtypeStruct((NW, ACC), jnp.float32),
        mesh=mesh,
        scratch_types=[
            pltpu.VMEM((ACC,), jnp.float32),
            pltpu.VMEM((2, CC), jnp.int32),
            pltpu.VMEM((CC,), jnp.float32),
            pltpu.VMEM((2, CC), jnp.int32),
            pltpu.VMEM((CC,), jnp.float32),
            pltpu.SemaphoreType.DMA,
            pltpu.SemaphoreType.DMA,
        ],
        compiler_params=pltpu.CompilerParams(needs_layout_passes=False),
    )
    def k(ei_hbm, y_hbm, out_hbm, acc, eb0, yb0, eb1, yb1, sem0, sem1):
        wid = lax.axis_index("s") * NC + lax.axis_index("c")
        n_c = (nchunks - wid + NW - 1) // NW
        sems = (sem0, sem1)
        bufs = ((eb0, yb0), (eb1, yb1))

        zeros = jnp.zeros((L,), jnp.float32)

        @plsc.parallel_loop(0, ACC // L, unroll=ZU)
        def _(i):
            acc[pl.ds(i * L, L)] = zeros

        def copies(j, b):
            c = wid + j * NW
            coff = pl.multiple_of(c * CC, 128)
            ebb, ybb = bufs[b]
            return (
                pltpu.make_async_copy(
                    ei_hbm.at[:, pl.ds(coff, CC)], ebb, sems[b]),
                pltpu.make_async_copy(
                    y_hbm.at[pl.ds(coff, CC)], ybb, sems[b]),
            )

        def issue(j, b):
            for cp in copies(j, b):
                cp.start()

        def wait(b):
            for cp in copies(0, b):
                cp.wait()

        row0 = jnp.zeros((L,), jnp.int32)
        row1 = jnp.ones((L,), jnp.int32)
        lanes = lax.iota(jnp.int32, L)

        def compute(b):
            ebb, ybb = bufs[b]

            @plsc.parallel_loop(0, CC // L, unroll=U)
            def _(i):
                col = lanes + i * L
                s = plsc.load_gather(ebb, [row0, col])
                d = plsc.load_gather(ebb, [row1, col])
                v = ybb[pl.ds(i * L, L)]
                plsc.addupdate_scatter(acc, [s], v)
                plsc.addupdate_scatter(acc, [d], -v)

        issue(0, 0)

        def pair_body(p, _):
            j0 = p * 2
            j1 = j0 + 1

            @pl.when(j1 < n_c)
            def _():
                issue(j1, 1)

            @pl.when(j0 < n_c)
            def _():
                wait(0)
                compute(0)

            @pl.when(j0 + 2 < n_c)
            def _():
                issue(j0 + 2, 0)

            @pl.when(j1 < n_c)
            def _():
                wait(1)
                compute(1)

            return 0

        lax.fori_loop(0, npairs, pair_body, 0)

        pltpu.sync_copy(acc, out_hbm.at[wid])

    return k(edge_index, yh)


def _tc_loss(partials):
    grid = 4
    blk = ACC // grid

    def body(p_ref, o_ref):
        @pl.when(pl.program_id(0) == 0)
        def _():
            o_ref[0, 0] = 0.0

        d = jnp.sum(p_ref[...], axis=0)
        o_ref[0, 0] += jnp.sum(jnp.abs(d))

    out = pl.pallas_call(
        body,
        grid=(grid,),
        in_specs=[pl.BlockSpec((NW, blk), lambda i: (0, i))],
        out_specs=pl.BlockSpec(memory_space=pltpu.SMEM),
        out_shape=jax.ShapeDtypeStruct((1, 1), jnp.float32),
    )(partials)
    return out[0, 0]


def kernel(edge_index, y_hat):
    yh = jnp.squeeze(y_hat, -1).astype(jnp.float32)
    partials = _sc_partials(edge_index, yh)
    total = _tc_loss(partials)
    return total / yh.shape[0]


# speedup vs baseline: 1.0658x; 1.0132x over previous
import functools

import jax
import jax.numpy as jnp
from jax import lax
from jax.experimental import pallas as pl
from jax.experimental.pallas import tpu as pltpu
from jax.experimental.pallas import tpu_sc as plsc

NC = 2
NS = 16
NW = NC * NS
L = 16

N_NODES = 100_000
ACC = 100_352
CC = 3200
U = 8
ZU = 32


def _sc_partials(edge_index, yh):
    (E,) = yh.shape
    nchunks = E // CC
    assert nchunks * CC == E
    npairs = (nchunks + NW - 1) // NW
    npairs = (npairs + 1) // 2

    mesh = plsc.VectorSubcoreMesh(core_axis_name="c", subcore_axis_name="s")

    @functools.partial(
        pl.kernel,
        out_type=jax.ShapeDtypeStruct((NW, ACC), jnp.float32),
        mesh=mesh,
        scratch_types=[
            pltpu.VMEM((ACC,), jnp.float32),
            pltpu.VMEM((2, CC), jnp.int32),
            pltpu.VMEM((CC,), jnp.float32),
            pltpu.VMEM((2, CC), jnp.int32),
            pltpu.VMEM((CC,), jnp.float32),
            pltpu.SemaphoreType.DMA,
            pltpu.SemaphoreType.DMA,
        ],
        compiler_params=pltpu.CompilerParams(needs_layout_passes=False),
    )
    def k(ei_hbm, y_hbm, out_hbm, acc, eb0, yb0, eb1, yb1, sem0, sem1):
        wid = lax.axis_index("s") * NC + lax.axis_index("c")
        n_c = (nchunks - wid + NW - 1) // NW
        sems = (sem0, sem1)
        bufs = ((eb0, yb0), (eb1, yb1))

        def copies(j, b):
            c = wid + j * NW
            coff = pl.multiple_of(c * CC, 128)
            ebb, ybb = bufs[b]
            return (
                pltpu.make_async_copy(
                    ei_hbm.at[:, pl.ds(coff, CC)], ebb, sems[b]),
                pltpu.make_async_copy(
                    y_hbm.at[pl.ds(coff, CC)], ybb, sems[b]),
            )

        def issue(j, b):
            for cp in copies(j, b):
                cp.start()

        def wait(b):
            for cp in copies(0, b):
                cp.wait()

        row0 = jnp.zeros((L,), jnp.int32)
        row1 = jnp.ones((L,), jnp.int32)
        lanes = lax.iota(jnp.int32, L)

        def compute(b):
            ebb, ybb = bufs[b]

            @plsc.parallel_loop(0, CC // L, unroll=U)
            def _(i):
                col = lanes + i * L
                s = plsc.load_gather(ebb, [row0, col])
                d = plsc.load_gather(ebb, [row1, col])
                v = ybb[pl.ds(i * L, L)]
                plsc.addupdate_scatter(acc, [s], v)
                plsc.addupdate_scatter(acc, [d], -v)

        issue(0, 0)

        zeros = jnp.zeros((L,), jnp.float32)

        @plsc.parallel_loop(0, ACC // L, unroll=ZU)
        def _(i):
            acc[pl.ds(i * L, L)] = zeros

        def pair_body(p, _):
            j0 = p * 2
            j1 = j0 + 1

            @pl.when(j1 < n_c)
            def _():
                issue(j1, 1)

            @pl.when(j0 < n_c)
            def _():
                wait(0)
                compute(0)

            @pl.when(j0 + 2 < n_c)
            def _():
                issue(j0 + 2, 0)

            @pl.when(j1 < n_c)
            def _():
                wait(1)
                compute(1)

            return 0

        lax.fori_loop(0, npairs, pair_body, 0)

        pltpu.sync_copy(acc, out_hbm.at[wid])

    return k(edge_index, yh)


def _tc_loss(partials):
    grid = 4
    blk = ACC // grid

    def body(p_ref, o_ref):
        @pl.when(pl.program_id(0) == 0)
        def _():
            o_ref[0, 0] = 0.0

        d = jnp.sum(p_ref[...], axis=0)
        o_ref[0, 0] += jnp.sum(jnp.abs(d))

    out = pl.pallas_call(
        body,
        grid=(grid,),
        in_specs=[pl.BlockSpec((NW, blk), lambda i: (0, i))],
        out_specs=pl.BlockSpec(memory_space=pltpu.SMEM),
        out_shape=jax.ShapeDtypeStruct((1, 1), jnp.float32),
    )(partials)
    return out[0, 0]


def kernel(edge_index, y_hat):
    yh = jnp.squeeze(y_hat, -1).astype(jnp.float32)
    partials = _sc_partials(edge_index, yh)
    total = _tc_loss(partials)
    return total / yh.shape[0]


# speedup vs baseline: 1.1218x; 1.0526x over previous
import functools

import jax
import jax.numpy as jnp
from jax import lax
from jax.experimental import pallas as pl
from jax.experimental.pallas import tpu as pltpu
from jax.experimental.pallas import tpu_sc as plsc

NC = 2
NS = 16
NW = NC * NS
L = 16

N_NODES = 100_000
ACC = 100_224
CC = 5120
U = 8
ZU = 32


def _sc_partials(edge_index, yh):
    (E,) = yh.shape
    nchunks = E // CC
    assert nchunks * CC == E
    npairs = (nchunks + NW - 1) // NW
    npairs = (npairs + 1) // 2

    mesh = plsc.VectorSubcoreMesh(core_axis_name="c", subcore_axis_name="s")

    @functools.partial(
        pl.kernel,
        out_type=jax.ShapeDtypeStruct((NW, ACC), jnp.float32),
        mesh=mesh,
        scratch_types=[
            pltpu.VMEM((ACC,), jnp.float32),
            pltpu.VMEM((2, CC), jnp.int32),
            pltpu.VMEM((CC,), jnp.float32),
            pltpu.VMEM((2, CC), jnp.int32),
            pltpu.VMEM((CC,), jnp.float32),
            pltpu.SemaphoreType.DMA,
            pltpu.SemaphoreType.DMA,
        ],
        compiler_params=pltpu.CompilerParams(needs_layout_passes=False),
    )
    def k(ei_hbm, y_hbm, out_hbm, acc, eb0, yb0, eb1, yb1, sem0, sem1):
        wid = lax.axis_index("s") * NC + lax.axis_index("c")
        n_c = (nchunks - wid + NW - 1) // NW
        sems = (sem0, sem1)
        bufs = ((eb0, yb0), (eb1, yb1))

        def copies(j, b):
            c = wid + j * NW
            coff = pl.multiple_of(c * CC, 128)
            ebb, ybb = bufs[b]
            return (
                pltpu.make_async_copy(
                    ei_hbm.at[:, pl.ds(coff, CC)], ebb, sems[b]),
                pltpu.make_async_copy(
                    y_hbm.at[pl.ds(coff, CC)], ybb, sems[b]),
            )

        def issue(j, b):
            for cp in copies(j, b):
                cp.start()

        def wait(b):
            for cp in copies(0, b):
                cp.wait()

        row0 = jnp.zeros((L,), jnp.int32)
        row1 = jnp.ones((L,), jnp.int32)
        lanes = lax.iota(jnp.int32, L)

        def compute(b):
            ebb, ybb = bufs[b]

            @plsc.parallel_loop(0, CC // L, unroll=U)
            def _(i):
                col = lanes + i * L
                s = plsc.load_gather(ebb, [row0, col])
                d = plsc.load_gather(ebb, [row1, col])
                v = ybb[pl.ds(i * L, L)]
                plsc.addupdate_scatter(acc, [s], v)
                plsc.addupdate_scatter(acc, [d], -v)

        issue(0, 0)

        zeros = jnp.zeros((L,), jnp.float32)

        @plsc.parallel_loop(0, ACC // L, unroll=ZU)
        def _(i):
            acc[pl.ds(i * L, L)] = zeros

        def pair_body(p, _):
            j0 = p * 2
            j1 = j0 + 1

            @pl.when(j1 < n_c)
            def _():
                issue(j1, 1)

            @pl.when(j0 < n_c)
            def _():
                wait(0)
                compute(0)

            @pl.when(j0 + 2 < n_c)
            def _():
                issue(j0 + 2, 0)

            @pl.when(j1 < n_c)
            def _():
                wait(1)
                compute(1)

            return 0

        lax.fori_loop(0, npairs, pair_body, 0)

        pltpu.sync_copy(acc, out_hbm.at[wid])

    return k(edge_index, yh)


def _tc_loss(partials):
    grid = 3
    blk = ACC // grid

    def body(p_ref, o_ref):
        @pl.when(pl.program_id(0) == 0)
        def _():
            o_ref[0, 0] = 0.0

        d = jnp.sum(p_ref[...], axis=0)
        o_ref[0, 0] += jnp.sum(jnp.abs(d))

    out = pl.pallas_call(
        body,
        grid=(grid,),
        in_specs=[pl.BlockSpec((NW, blk), lambda i: (0, i))],
        out_specs=pl.BlockSpec(memory_space=pltpu.SMEM),
        out_shape=jax.ShapeDtypeStruct((1, 1), jnp.float32),
    )(partials)
    return out[0, 0]


def kernel(edge_index, y_hat):
    yh = jnp.squeeze(y_hat, -1).astype(jnp.float32)
    partials = _sc_partials(edge_index, yh)
    total = _tc_loss(partials)
    return total / yh.shape[0]
